# speedup vs baseline: 1.1208x; 1.1208x over previous
import functools

import jax
import jax.numpy as jnp
import numpy as np
from jax.experimental import pallas as pl

_AVG_DEG_LOG = float(np.log(33.0))


def _matmul_kernel(x_ref, w_ref, b_ref, o_ref):
    o_ref[...] = (
        jnp.dot(x_ref[...], w_ref[...], preferred_element_type=jnp.float32)
        + b_ref[...]
    )


def _rows_matmul(x, w, b, blk):
    m, k = x.shape
    n = w.shape[1]
    grid = (m // blk,)
    return pl.pallas_call(
        _matmul_kernel,
        grid=grid,
        in_specs=[
            pl.BlockSpec((blk, k), lambda i: (i, 0)),
            pl.BlockSpec((k, n), lambda i: (0, 0)),
            pl.BlockSpec((1, n), lambda i: (0, 0)),
        ],
        out_specs=pl.BlockSpec((blk, n), lambda i: (i, 0)),
        out_shape=jax.ShapeDtypeStruct((m, n), jnp.float32),
    )(x, w, b)


def _post_kernel(x_ref, t1_ref, t2_ref, tmx_ref, tmn_ref, cnt_ref,
                 wd_ref, wf_ref, bf_ref, o_ref):
    f = x_ref.shape[1]
    x = x_ref[...]
    cnt_raw = cnt_ref[...]
    has = cnt_raw > 0.0
    cnt = jnp.maximum(cnt_raw, 1.0)
    xd = jnp.dot(x, wd_ref[...], preferred_element_type=jnp.float32)
    m1 = t1_ref[...] / cnt
    mean = jnp.where(has, xd + m1, 0.0)
    var = t2_ref[...] / cnt - m1 * m1
    std = jnp.sqrt(jnp.maximum(var, 0.0) + 1e-5)
    mx = jnp.where(has, xd + tmx_ref[...], 0.0)
    mn = jnp.where(has, xd + tmn_ref[...], 0.0)
    log_deg = jnp.log(cnt + 1.0)
    amp = log_deg / _AVG_DEG_LOG
    att = _AVG_DEG_LOG / log_deg

    wf = wf_ref[...]

    def dot_w(a, j):
        return jnp.dot(a, wf[j * f:(j + 1) * f, :],
                       preferred_element_type=jnp.float32)

    aggs = (mean, mn, mx, std)
    acc = dot_w(x, 0) + bf_ref[...]
    for a_i, a in enumerate(aggs):
        acc += dot_w(a, 1 + a_i)
    acc += amp * sum(dot_w(a, 5 + a_i) for a_i, a in enumerate(aggs))
    acc += att * sum(dot_w(a, 9 + a_i) for a_i, a in enumerate(aggs))
    o_ref[...] = acc


def kernel(x, edge_index, edge_attr, W_edge, b_edge, W_pre, b_pre,
           W_post, b_post, W_lin, b_lin):
    n, f = x.shape
    e = edge_index.shape[1]

    wd = W_pre[:f]
    ws = W_pre[f:2 * f]
    we = W_edge @ W_pre[2 * f:]
    be = (b_edge @ W_pre[2 * f:] + b_pre)[None, :]
    wf = W_post @ W_lin
    bf = (b_post @ W_lin + b_lin)[None, :]

    xs = _rows_matmul(x, ws, jnp.zeros_like(be), 1000)
    ep = _rows_matmul(edge_attr, we, be, 2000)

    src = edge_index[0]
    dst = edge_index[1]
    g = jnp.take(xs, src, axis=0) + ep
    t1 = jax.ops.segment_sum(g, dst, num_segments=n)
    t2 = jax.ops.segment_sum(g * g, dst, num_segments=n)
    tmx = jax.ops.segment_max(g, dst, num_segments=n)
    tmn = -jax.ops.segment_max(-g, dst, num_segments=n)
    cnt = jax.ops.segment_sum(jnp.ones((e,), jnp.float32), dst,
                              num_segments=n)[:, None]

    blk = 1000
    out = pl.pallas_call(
        _post_kernel,
        grid=(n // blk,),
        in_specs=[
            pl.BlockSpec((blk, f), lambda i: (i, 0)),
            pl.BlockSpec((blk, f), lambda i: (i, 0)),
            pl.BlockSpec((blk, f), lambda i: (i, 0)),
            pl.BlockSpec((blk, f), lambda i: (i, 0)),
            pl.BlockSpec((blk, f), lambda i: (i, 0)),
            pl.BlockSpec((blk, 1), lambda i: (i, 0)),
            pl.BlockSpec((f, f), lambda i: (0, 0)),
            pl.BlockSpec((13 * f, f), lambda i: (0, 0)),
            pl.BlockSpec((1, f), lambda i: (0, 0)),
        ],
        out_specs=pl.BlockSpec((blk, f), lambda i: (i, 0)),
        out_shape=jax.ShapeDtypeStruct((n, f), jnp.float32),
    )(x, t1, t2, tmx, tmn, cnt, wd, wf, bf)
    return out


# speedup vs baseline: 1.1210x; 1.0002x over previous
import functools

import jax
import jax.numpy as jnp
import numpy as np
from jax import lax
from jax.experimental import pallas as pl
from jax.experimental.pallas import tpu as pltpu
from jax.experimental.pallas import tpu_sc as plsc

_AVG_DEG_LOG = float(np.log(33.0))


def _mm_kernel(x_ref, w_ref, b_ref, o_ref):
    o_ref[...] = (
        jnp.dot(x_ref[...], w_ref[...], preferred_element_type=jnp.float32)
        + b_ref[...]
    )


def _rows_matmul(x, w, b, blk):
    m, k = x.shape
    nn = w.shape[1]
    return pl.pallas_call(
        _mm_kernel,
        grid=(m // blk,),
        in_specs=[
            pl.BlockSpec((blk, k), lambda i: (i, 0)),
            pl.BlockSpec((k, nn), lambda i: (0, 0)),
            pl.BlockSpec((1, nn), lambda i: (0, 0)),
        ],
        out_specs=pl.BlockSpec((blk, nn), lambda i: (i, 0)),
        out_shape=jax.ShapeDtypeStruct((m, nn), jnp.float32),
    )(x, w, b)


def _split_mm_kernel(x_ref, w_ref, b_ref, o_ref):
    o_ref[...] = (
        jnp.dot(x_ref[...], w_ref[0], preferred_element_type=jnp.float32)
        + b_ref[0]
    )


def _split_matmul(x, w, b, blk):
    m, k = x.shape
    h = w.shape[1] // 2
    nb = m // blk
    w2 = jnp.stack([w[:, :h], w[:, h:]])
    b2 = jnp.stack([b[:, :h], b[:, h:]])
    return pl.pallas_call(
        _split_mm_kernel,
        grid=(nb, 2),
        in_specs=[
            pl.BlockSpec((blk, k), lambda i, j: (i, 0)),
            pl.BlockSpec((1, k, h), lambda i, j: (j, 0, 0)),
            pl.BlockSpec((1, 1, h), lambda i, j: (j, 0, 0)),
        ],
        out_specs=pl.BlockSpec((blk, h), lambda i, j: (j * nb + i, 0)),
        out_shape=jax.ShapeDtypeStruct((2 * m, h), jnp.float32),
    )(x, w2, b2)


def _seg_sums_sc(dst, src, xs, epflat, n, e, h):
    chunk = 64
    q4 = h // 2
    nchunks = e // chunk
    iters = (nchunks + 15) // 16
    mesh = plsc.VectorSubcoreMesh(core_axis_name="c", subcore_axis_name="s")

    @functools.partial(
        pl.kernel,
        out_type=(
            jax.ShapeDtypeStruct((4 * n, q4), jnp.float32),
            jax.ShapeDtypeStruct((4 * n, q4), jnp.float32),
        ),
        mesh=mesh,
        scratch_types=(
            pltpu.VMEM((chunk,), jnp.int32),
            pltpu.VMEM((chunk,), jnp.int32),
            pltpu.VMEM((chunk, 2 * h), jnp.float32),
            pltpu.VMEM((chunk * h,), jnp.float32),
            pltpu.VMEM((chunk, q4), jnp.float32),
            pltpu.VMEM((chunk, q4), jnp.float32),
            pltpu.VMEM((40, q4), jnp.float32),
            pltpu.VMEM_SHARED((n, q4), jnp.float32),
            pltpu.VMEM_SHARED((n, q4), jnp.float32),
            pltpu.SemaphoreType.DMA,
            pltpu.SemaphoreType.DMA,
            pltpu.SemaphoreType.DMA,
            pltpu.SemaphoreType.DMA,
            pltpu.SemaphoreType.DMA,
            pltpu.SemaphoreType.DMA,
        ),
    )
    def body(dst_hbm, src_hbm, xs_hbm, ep_hbm, t1_hbm, t2_hbm,
             dstv, srcv, xsb, epb, gb, g2b, zb, t1_sp, t2_sp,
             sem0, sem1, sem2, sem3, sem4, sem5):
        c = lax.axis_index("c")
        s = lax.axis_index("s")

        @pl.loop(0, 40)
        def _init_zb(r):
            for v in range(q4 // 16):
                zb[r, pl.ds(v * 16, 16)] = jnp.zeros((16,), jnp.float32)

        ngroups = n // 40
        giters = (ngroups + 15) // 16

        @pl.loop(0, 2)
        def _qpass(qp):
            @pl.loop(0, giters)
            def _zero(j):
                gidx = s + 16 * j

                @pl.when(gidx < ngroups)
                def _():
                    r0 = gidx * 40
                    pltpu.sync_copy(zb, t1_sp.at[pl.ds(r0, 40)])
                    pltpu.sync_copy(zb, t2_sp.at[pl.ds(r0, 40)])

            plsc.subcore_barrier()

            @pl.loop(0, iters)
            def _chunk(j):
                ci = s + 16 * j

                @pl.when(ci < nchunks)
                def _():
                    base = ci * chunk
                    cd = pltpu.async_copy(dst_hbm.at[pl.ds(base, chunk)],
                                          dstv, sem0)
                    cs = pltpu.async_copy(src_hbm.at[pl.ds(base, chunk)],
                                          srcv, sem1)
                    ce = pltpu.async_copy(
                        ep_hbm.at[pl.ds((qp * e + base) * h, chunk * h)],
                        epb, sem2)
                    cs.wait()
                    cg = pltpu.async_copy(xs_hbm.at[srcv], xsb, sem3)
                    ce.wait()
                    cg.wait()

                    @pl.loop(0, chunk)
                    def _rows(r):
                        for v in range(q4 // 16):
                            sl = pl.ds(v * 16, 16)
                            g = (xsb[r, pl.ds(qp * h + c * q4 + v * 16, 16)]
                                 + epb[pl.ds(r * h + c * q4 + v * 16, 16)])
                            gb[r, sl] = g
                            g2b[r, sl] = g * g

                    cd.wait()
                    s1 = pltpu.async_copy(gb, t1_sp.at[dstv], sem4,
                                          add=True)
                    s2 = pltpu.async_copy(g2b, t2_sp.at[dstv], sem5,
                                          add=True)
                    s1.wait()
                    s2.wait()

            plsc.subcore_barrier()

            @pl.loop(0, giters)
            def _out(j):
                gidx = s + 16 * j

                @pl.when(gidx < ngroups)
                def _():
                    r0 = gidx * 40
                    o0 = (2 * qp + c) * n + r0
                    pltpu.sync_copy(t1_sp.at[pl.ds(r0, 40)],
                                    t1_hbm.at[pl.ds(o0, 40)])
                    pltpu.sync_copy(t2_sp.at[pl.ds(r0, 40)],
                                    t2_hbm.at[pl.ds(o0, 40)])

            plsc.subcore_barrier()

    return body(dst, src, xs, epflat)


def _post_kernel(x_ref, t1_ref, t2_ref,
                 tmx_ref, tmn_ref, cnt_ref, wd_ref, wf_ref, bf_ref, o_ref):
    f = x_ref.shape[1]
    x = x_ref[...]
    cnt_raw = cnt_ref[...]
    has = cnt_raw > 0.0
    cnt = jnp.maximum(cnt_raw, 1.0)
    t1 = t1_ref[...]
    t2 = t2_ref[...]
    xd = jnp.dot(x, wd_ref[...], preferred_element_type=jnp.float32)
    m1 = t1 / cnt
    mean = jnp.where(has, xd + m1, 0.0)
    var = t2 / cnt - m1 * m1
    std = jnp.sqrt(jnp.maximum(var, 0.0) + 1e-5)
    mx = jnp.where(has, xd + tmx_ref[...], 0.0)
    mn = jnp.where(has, xd + tmn_ref[...], 0.0)
    log_deg = jnp.log(cnt + 1.0)
    amp = log_deg / _AVG_DEG_LOG
    att = _AVG_DEG_LOG / log_deg

    wf = wf_ref[...]

    def dot_w(a, j):
        return jnp.dot(a, wf[j * f:(j + 1) * f, :],
                       preferred_element_type=jnp.float32)

    aggs = (mean, mn, mx, std)
    acc = dot_w(x, 0) + bf_ref[...]
    for a_i, a in enumerate(aggs):
        acc += dot_w(a, 1 + a_i)
    acc += amp * sum(dot_w(a, 5 + a_i) for a_i, a in enumerate(aggs))
    acc += att * sum(dot_w(a, 9 + a_i) for a_i, a in enumerate(aggs))
    o_ref[...] = acc


def kernel(x, edge_index, edge_attr, W_edge, b_edge, W_pre, b_pre,
           W_post, b_post, W_lin, b_lin):
    n, f = x.shape
    e = edge_index.shape[1]
    h = f // 2

    wd = W_pre[:f]
    ws = W_pre[f:2 * f]
    we = W_edge @ W_pre[2 * f:]
    be = (b_edge @ W_pre[2 * f:] + b_pre)[None, :]
    wf = W_post @ W_lin
    bf = (b_post @ W_lin + b_lin)[None, :]

    xs = _rows_matmul(x, ws, jnp.zeros_like(be), 1000)
    ep = _rows_matmul(edge_attr, we, be, 2000)

    src = edge_index[0]
    dst = edge_index[1]

    g = jnp.take(xs, src, axis=0) + ep
    t1 = jax.ops.segment_sum(g, dst, num_segments=n)
    t2 = jax.ops.segment_sum(g * g, dst, num_segments=n)
    cnt = jax.ops.segment_sum(jnp.ones((e,), jnp.float32), dst,
                              num_segments=n)[:, None]
    tmx = jax.ops.segment_max(g, dst, num_segments=n)
    tmn = -jax.ops.segment_max(-g, dst, num_segments=n)

    blk = 1000
    nb = n // blk
    out = pl.pallas_call(
        _post_kernel,
        grid=(nb,),
        in_specs=[
            pl.BlockSpec((blk, f), lambda i: (i, 0)),
            pl.BlockSpec((blk, f), lambda i: (i, 0)),
            pl.BlockSpec((blk, f), lambda i: (i, 0)),
            pl.BlockSpec((blk, f), lambda i: (i, 0)),
            pl.BlockSpec((blk, f), lambda i: (i, 0)),
            pl.BlockSpec((blk, 1), lambda i: (i, 0)),
            pl.BlockSpec((f, f), lambda i: (0, 0)),
            pl.BlockSpec((13 * f, f), lambda i: (0, 0)),
            pl.BlockSpec((1, f), lambda i: (0, 0)),
        ],
        out_specs=pl.BlockSpec((blk, f), lambda i: (i, 0)),
        out_shape=jax.ShapeDtypeStruct((n, f), jnp.float32),
    )(x, t1, t2, tmx, tmn, cnt, wd, wf, bf)
    return out


# speedup vs baseline: 1.1215x; 1.0005x over previous
import jax
import jax.numpy as jnp
import numpy as np
from jax.experimental import pallas as pl

_AVG_DEG_LOG = float(np.log(33.0))


def _mm_kernel(x_ref, w_ref, b_ref, o_ref):
    o_ref[...] = (
        jnp.dot(x_ref[...], w_ref[...], preferred_element_type=jnp.float32)
        + b_ref[...]
    )


def _rows_matmul(x, w, b, blk):
    m, k = x.shape
    nn = w.shape[1]
    return pl.pallas_call(
        _mm_kernel,
        grid=(m // blk,),
        in_specs=[
            pl.BlockSpec((blk, k), lambda i: (i, 0)),
            pl.BlockSpec((k, nn), lambda i: (0, 0)),
            pl.BlockSpec((1, nn), lambda i: (0, 0)),
        ],
        out_specs=pl.BlockSpec((blk, nn), lambda i: (i, 0)),
        out_shape=jax.ShapeDtypeStruct((m, nn), jnp.float32),
    )(x, w, b)


def _post_kernel(x_ref, t1_ref, t2_ref,
                 tmx_ref, tmn_ref, cnt_ref, wd_ref, wf_ref, bf_ref, o_ref):
    f = x_ref.shape[1]
    x = x_ref[...]
    cnt_raw = cnt_ref[...]
    has = cnt_raw > 0.0
    cnt = jnp.maximum(cnt_raw, 1.0)
    t1 = t1_ref[...]
    t2 = t2_ref[...]
    xd = jnp.dot(x, wd_ref[...], preferred_element_type=jnp.float32)
    m1 = t1 / cnt
    mean = jnp.where(has, xd + m1, 0.0)
    var = t2 / cnt - m1 * m1
    std = jnp.sqrt(jnp.maximum(var, 0.0) + 1e-5)
    mx = jnp.where(has, xd + tmx_ref[...], 0.0)
    mn = jnp.where(has, xd + tmn_ref[...], 0.0)
    log_deg = jnp.log(cnt + 1.0)
    amp = log_deg / _AVG_DEG_LOG
    att = _AVG_DEG_LOG / log_deg

    wf = wf_ref[...]

    def dot_w(a, j):
        return jnp.dot(a, wf[j * f:(j + 1) * f, :],
                       preferred_element_type=jnp.float32)

    aggs = (mean, mn, mx, std)
    acc = dot_w(x, 0) + bf_ref[...]
    for a_i, a in enumerate(aggs):
        acc += dot_w(a, 1 + a_i)
    acc += amp * sum(dot_w(a, 5 + a_i) for a_i, a in enumerate(aggs))
    acc += att * sum(dot_w(a, 9 + a_i) for a_i, a in enumerate(aggs))
    o_ref[...] = acc


def kernel(x, edge_index, edge_attr, W_edge, b_edge, W_pre, b_pre,
           W_post, b_post, W_lin, b_lin):
    n, f = x.shape
    e = edge_index.shape[1]
    h = f // 2

    wd = W_pre[:f]
    ws = W_pre[f:2 * f]
    we = W_edge @ W_pre[2 * f:]
    be = (b_edge @ W_pre[2 * f:] + b_pre)[None, :]
    wf = W_post @ W_lin
    bf = (b_post @ W_lin + b_lin)[None, :]

    xs = _rows_matmul(x, ws, jnp.zeros_like(be), 1000)
    ep = _rows_matmul(edge_attr, we, be, 2000)

    src = edge_index[0]
    dst = edge_index[1]

    g = jnp.take(xs, src, axis=0) + ep
    t1 = jax.ops.segment_sum(g, dst, num_segments=n)
    t2 = jax.ops.segment_sum(g * g, dst, num_segments=n)
    cnt = jax.ops.segment_sum(jnp.ones((e,), jnp.float32), dst,
                              num_segments=n)[:, None]
    tmx = jax.ops.segment_max(g, dst, num_segments=n)
    tmn = -jax.ops.segment_max(-g, dst, num_segments=n)

    blk = 1000
    nb = n // blk
    out = pl.pallas_call(
        _post_kernel,
        grid=(nb,),
        in_specs=[
            pl.BlockSpec((blk, f), lambda i: (i, 0)),
            pl.BlockSpec((blk, f), lambda i: (i, 0)),
            pl.BlockSpec((blk, f), lambda i: (i, 0)),
            pl.BlockSpec((blk, f), lambda i: (i, 0)),
            pl.BlockSpec((blk, f), lambda i: (i, 0)),
            pl.BlockSpec((blk, 1), lambda i: (i, 0)),
            pl.BlockSpec((f, f), lambda i: (0, 0)),
            pl.BlockSpec((13 * f, f), lambda i: (0, 0)),
            pl.BlockSpec((1, f), lambda i: (0, 0)),
        ],
        out_specs=pl.BlockSpec((blk, f), lambda i: (i, 0)),
        out_shape=jax.ShapeDtypeStruct((n, f), jnp.float32),
    )(x, t1, t2, tmx, tmn, cnt, wd, wf, bf)
    return out
